# baseline (speedup 1.0000x reference)
import functools

import jax
import jax.numpy as jnp
from jax import lax
from jax.experimental import pallas as pl
from jax.experimental.pallas import tpu as pltpu
from jax.experimental.pallas import tpu_sc as plsc

N_FIELDS = 26
VOCAB = 100000
N_DIM = 32
BATCH = 4096
L = 200

NC = 2
NS = 16
NW = NC * NS
B_PER_W = BATCH // NW
N_LINES = N_FIELDS * VOCAB // 4
LF = 8
NF_IT = L // LF


def _extract_plane(lines, offs, ttbuf):
    def do_c(c, _):
        for g in range(8):
            row = jnp.arange(16, dtype=jnp.int32) + (g * 16)
            col = offs.at[pl.ds(g * 16, 16)][...] + c
            v = plsc.load_gather(lines, [row, col])
            ttbuf.at[c, pl.ds(g * 16, 16)][...] = v
        return 0

    lax.fori_loop(0, N_DIM, do_c, 0, unroll=False)


def _body(tab_hbm, idx_hbm, off_hbm, out_hbm,
          idx_v, off_v, l0, l1, tt0, tt1,
          g0sem, g1sem, t0sem, t1sem):
    wid = lax.axis_index("s") * NC + lax.axis_index("c")
    b0 = wid * B_PER_W
    pltpu.sync_copy(idx_hbm.at[wid], idx_v)
    pltpu.sync_copy(off_hbm.at[wid], off_v)

    lbufs = (l0, l1)
    tbufs = (tt0, tt1)
    gsems = (g0sem, g1sem)
    tsems = (t0sem, t1sem)

    def gfire(f):
        return pltpu.async_copy(
            tab_hbm.at[idx_v.at[f]], lbufs[f % 2], gsems[f % 2])

    def tfire(f):
        return pltpu.async_copy(
            tbufs[f % 2], out_hbm.at[f, pl.ds(0, N_DIM), pl.ds(b0, B_PER_W)],
            tsems[f % 2])

    gpend = gfire(0)
    tpend = [None, None]

    for f in range(N_FIELDS):
        if f + 1 < N_FIELDS:
            gnext = gfire(f + 1)
        gpend.wait()
        if tpend[f % 2] is not None:
            tpend[f % 2].wait()
            tpend[f % 2] = None
        _extract_plane(lbufs[f % 2], off_v.at[f], tbufs[f % 2])
        tpend[f % 2] = tfire(f)
        if f + 1 < N_FIELDS:
            gpend = gnext

    for h in tpend:
        if h is not None:
            h.wait()


_fused = functools.partial(
    pl.kernel,
    mesh=plsc.VectorSubcoreMesh(core_axis_name="c", subcore_axis_name="s"),
    compiler_params=pltpu.CompilerParams(
        use_tc_tiling_on_sc=True, needs_layout_passes=False),
    out_type=jax.ShapeDtypeStruct((N_FIELDS, N_DIM, BATCH), jnp.float32),
    scratch_types=[
        pltpu.VMEM((N_FIELDS, B_PER_W), jnp.int32),
        pltpu.VMEM((N_FIELDS, B_PER_W), jnp.int32),
        pltpu.VMEM((B_PER_W, 128), jnp.float32),
        pltpu.VMEM((B_PER_W, 128), jnp.float32),
        pltpu.VMEM((N_DIM, B_PER_W), jnp.float32),
        pltpu.VMEM((N_DIM, B_PER_W), jnp.float32),
    ] + [pltpu.SemaphoreType.DMA] * 4,
)(_body)


def kernel(feature, indices, tables):
    tab_lines = tables.reshape(N_LINES, 128)
    flat = indices.astype(jnp.int32) + (
        jnp.arange(N_FIELDS, dtype=jnp.int32) * VOCAB)[None, :]
    flat = flat.T.reshape(N_FIELDS, NW, B_PER_W).transpose(1, 0, 2)
    lines = flat // 4
    offs = (flat % 4) * N_DIM
    emb_t = _fused(tab_lines, lines, offs)
    embeds = jnp.transpose(emb_t, (2, 0, 1))
    return jnp.concatenate([embeds, feature], axis=1)


# speedup vs baseline: 1.0457x; 1.0457x over previous
import functools

import jax
import jax.numpy as jnp
from jax import lax
from jax.experimental import pallas as pl
from jax.experimental.pallas import tpu as pltpu
from jax.experimental.pallas import tpu_sc as plsc

N_FIELDS = 26
VOCAB = 100000
N_DIM = 32
BATCH = 4096
L = 200

NC = 2
NS = 16
NW = NC * NS
B_FLAT = BATCH * N_FIELDS
ROWS_PER_W = B_FLAT // NW
CHUNK = 128
N_CHUNKS = ROWS_PER_W // CHUNK


def _gather_body(tab_hbm, idx_hbm, out_hbm, idx_v, rows_v, sem):
    wid = lax.axis_index("s") * NC + lax.axis_index("c")
    pltpu.sync_copy(idx_hbm.at[wid], idx_v)
    handles = []
    for j in range(N_CHUNKS):
        handles.append(
            pltpu.async_copy(
                tab_hbm.at[idx_v.at[j]],
                rows_v.at[pl.ds(j * CHUNK, CHUNK)],
                sem,
            )
        )
    for h in handles:
        h.wait()
    pltpu.sync_copy(rows_v, out_hbm.at[pl.ds(wid * ROWS_PER_W, ROWS_PER_W)])


_gather = functools.partial(
    pl.kernel,
    mesh=plsc.VectorSubcoreMesh(core_axis_name="c", subcore_axis_name="s"),
    compiler_params=pltpu.CompilerParams(use_tc_tiling_on_sc=False),
    out_type=jax.ShapeDtypeStruct((B_FLAT, N_DIM), jnp.float32),
    scratch_types=[
        pltpu.VMEM((N_CHUNKS, CHUNK), jnp.int32),
        pltpu.VMEM((ROWS_PER_W, N_DIM), jnp.float32),
        pltpu.SemaphoreType.DMA,
    ],
)(_gather_body)


def kernel(feature, indices, tables):
    tab_flat = tables.reshape(N_FIELDS * VOCAB, N_DIM)
    flat_idx = indices.astype(jnp.int32) + (
        jnp.arange(N_FIELDS, dtype=jnp.int32) * VOCAB
    )[None, :]
    flat_idx = flat_idx.reshape(NW, N_CHUNKS, CHUNK)
    embeds = _gather(tab_flat, flat_idx).reshape(BATCH, N_FIELDS, N_DIM)
    return jnp.concatenate([embeds, feature], axis=1)
